# baseline (speedup 1.0000x reference)
import jax
import jax.numpy as jnp
from jax.experimental import pallas as pl

B = 2048
N = 5
D = 64
H = 64
A = 16
PAIRS = [(i, j) for i in range(N) for j in range(N) if i != j]
P = len(PAIRS)

BB = 512


def _gnn_kernel(states_ref, act_ref, We_ref, be_ref, Wn_ref, bn_ref, out_ref):
    s = states_ref[...]
    a = act_ref[...]

    s_bf = [s[:, n * D:(n + 1) * D].astype(jnp.bfloat16) for n in range(N)]
    a_bf = [a[:, n * A:(n + 1) * A].astype(jnp.bfloat16) for n in range(N)]

    agg = [None] * N
    for p, (i, j) in enumerate(PAIRS):
        pre = (
            jnp.dot(s_bf[i], We_ref[p, :D], preferred_element_type=jnp.float32)
            + jnp.dot(s_bf[j], We_ref[p, D:], preferred_element_type=jnp.float32)
            + be_ref[p]
        )
        m = jnp.tanh(pre)
        agg[i] = m if agg[i] is None else agg[i] + m

    for n in range(N):
        pre = (
            jnp.dot(s_bf[n], Wn_ref[n, :D], preferred_element_type=jnp.float32)
            + jnp.dot(a_bf[n], Wn_ref[n, D:D + A], preferred_element_type=jnp.float32)
            + jnp.dot(agg[n].astype(jnp.bfloat16), Wn_ref[n, D + A:],
                      preferred_element_type=jnp.float32)
            + bn_ref[n]
        )
        out_ref[:, n * D:(n + 1) * D] = jnp.tanh(pre)


def kernel(states, action_vec, W_edge, b_edge, W_node, b_node):
    s2 = states.reshape(B, N * D)
    a2 = action_vec.reshape(B, N * A)
    We_bf = W_edge.astype(jnp.bfloat16)
    Wn_bf = W_node.astype(jnp.bfloat16)
    grid = (B // BB,)
    out = pl.pallas_call(
        _gnn_kernel,
        grid=grid,
        in_specs=[
            pl.BlockSpec((BB, N * D), lambda g: (g, 0)),
            pl.BlockSpec((BB, N * A), lambda g: (g, 0)),
            pl.BlockSpec((P, 2 * D, H), lambda g: (0, 0, 0)),
            pl.BlockSpec((P, H), lambda g: (0, 0)),
            pl.BlockSpec((N, D + A + H, D), lambda g: (0, 0, 0)),
            pl.BlockSpec((N, D), lambda g: (0, 0)),
        ],
        out_specs=pl.BlockSpec((BB, N * D), lambda g: (g, 0)),
        out_shape=jax.ShapeDtypeStruct((B, N * D), jnp.float32),
    )(s2, a2, We_bf, b_edge, Wn_bf, b_node)
    return out.reshape(B, N, D)
